# Initial kernel scaffold; baseline (speedup 1.0000x reference)
import jax
import jax.numpy as jnp
from jax.experimental import pallas as pl


def kernel(x, params):
    raise NotImplementedError("write your pallas kernel here")


# speedup vs baseline: 1.8520x; 1.8520x over previous
import itertools

import numpy as np
import jax
import jax.numpy as jnp
from jax.experimental import pallas as pl
from jax.experimental.pallas import tpu as pltpu

_EPS = 1e-5


def _geom(n, d, h, w, p):
    wp = -(-(w + 2 * p) // 8) * 8
    pw0 = (wp - w) // 2
    pw1 = wp - w - pw0
    dp, hp = d + 2 * p, h + 2 * p
    mp = n * dp * hp * wp
    e0t = p * (hp * wp + wp + 1)
    e0 = -(-e0t // 8) * 8
    blk = 512 if mp > 512 else -(-mp // 8) * 8
    mpr = -(-mp // blk) * blk
    return dict(n=n, d=d, h=h, w=w, p=p, dp=dp, hp=hp, wp=wp,
                pw0=pw0, pw1=pw1, mp=mp, e0=e0, delta=e0 - e0t,
                m=n * d * h * w, blk=blk, mpr=mpr, nb=mpr // blk)


def _pad_flat_ext(x5, g):
    p = g['p']
    xp = jnp.pad(x5, ((0, 0), (p, p), (p, p), (g['pw0'], g['pw1']), (0, 0)))
    xf = xp.reshape(g['mp'], x5.shape[-1])
    return jnp.pad(xf, ((g['e0'], g['mpr'] - g['mp'] + g['e0'] + 8), (0, 0)))


def _unpad5_ext(zf, g, c):
    z5 = zf[g['e0']:g['e0'] + g['mp']].reshape(
        g['n'], g['dp'], g['hp'], g['wp'], c)
    p = g['p']
    return z5[:, p:p + g['d'], p:p + g['h'], g['pw0']:g['pw0'] + g['w'], :]


def _mask_const(g):
    m5 = np.zeros((g['n'], g['dp'], g['hp'], g['wp'], 1), np.float32)
    p = g['p']
    m5[:, p:p + g['d'], p:p + g['h'], g['pw0']:g['pw0'] + g['w'], :] = 1.0
    mf = np.zeros((g['mpr'], 1), np.float32)
    mf[:g['mp']] = m5.reshape(g['mp'], 1)
    return jnp.asarray(mf)


def _zero_margins(ref, g, c):
    e0 = g['e0']
    ref[pl.ds(0, e0), :] = jnp.zeros((e0, c), jnp.float32)
    ref[pl.ds(e0 + g['mpr'], e0 + 8), :] = jnp.zeros((e0 + 8, c), jnp.float32)


def _conv_pass(src_ref, w_ref, m_ref, dst_ref, g, k, dst_ext):
    e0, blk, hw = g['e0'], g['blk'], g['hp'] * g['wp']
    co = w_ref.shape[-1]
    doff = e0 if dst_ext else 0

    ci = src_ref.shape[-1]

    def body(i, carry):
        s, comp = carry
        base = pl.multiple_of(i * blk, 8)
        acc = jnp.zeros((blk, co), jnp.float32)
        for td, th, tw in itertools.product(range(k), repeat=3):
            q = td * hw + th * g['wp'] + tw + g['delta']
            q_al, sub = q - q % 8, q % 8
            xs = src_ref[pl.ds(base + q_al, blk + 8), :]
            xs = jax.lax.slice(xs, (sub, 0), (sub + blk, ci))
            acc = acc + jnp.dot(xs, w_ref[td, th, tw],
                                preferred_element_type=jnp.float32)
        mt = m_ref[pl.ds(base, blk), :]
        accm = acc * mt
        dst_ref[pl.ds(doff + base, blk), :] = accm
        y = jnp.sum(accm, axis=0, keepdims=True) - comp
        t = s + y
        return t, (t - s) - y

    z = jnp.zeros((1, co), jnp.float32)
    return jax.lax.fori_loop(0, g['nb'], body, (z, z))[0]


def _var_pass(src_ref, m_ref, g, mean, src_ext):
    blk = g['blk']
    soff = g['e0'] if src_ext else 0

    def body(i, carry):
        ss, comp = carry
        base = i * blk
        cen = (src_ref[pl.ds(soff + base, blk), :] - mean) * \
            m_ref[pl.ds(base, blk), :]
        y = jnp.sum(cen * cen, axis=0, keepdims=True) - comp
        t = ss + y
        return t, (t - ss) - y

    z = jnp.zeros((1, src_ref.shape[-1]), jnp.float32)
    return jax.lax.fori_loop(0, g['nb'], body, (z, z))[0]


def _mm_pass(src_ref, w2_ref, m_ref, dst_ref, g, src_ext):
    blk = g['blk']
    co = w2_ref.shape[-1]
    soff = g['e0'] if src_ext else 0

    def body(i, carry):
        s, comp = carry
        base = i * blk
        t = jnp.dot(src_ref[pl.ds(soff + base, blk), :], w2_ref[0, 0, 0],
                    preferred_element_type=jnp.float32)
        accm = t * m_ref[pl.ds(base, blk), :]
        dst_ref[pl.ds(base, blk), :] = accm
        y = jnp.sum(accm, axis=0, keepdims=True) - comp
        u = s + y
        return u, (u - s) - y

    z = jnp.zeros((1, co), jnp.float32)
    return jax.lax.fori_loop(0, g['nb'], body, (z, z))[0]


def _norm_pass(src_ref, m_ref, dst_ref, g, mean, r, relu, src_ext, dst_ext,
               add_ref=None, add_stats=None, add_ext=False):
    blk = g['blk']
    soff = g['e0'] if src_ext else 0
    doff = g['e0'] if dst_ext else 0
    aoff = g['e0'] if add_ext else 0

    def body(i, _):
        base = i * blk
        mt = m_ref[pl.ds(base, blk), :]
        t = (src_ref[pl.ds(soff + base, blk), :] - mean) / r * mt
        if add_ref is not None:
            if add_stats is None:
                t = t + add_ref[pl.ds(aoff + base, blk), :]
            else:
                am, ar = add_stats
                t = t + (add_ref[pl.ds(aoff + base, blk), :] - am) / ar * mt
        if relu:
            t = jnp.maximum(t, 0.0)
        dst_ref[pl.ds(doff + base, blk), :] = t
        return 0

    jax.lax.fori_loop(0, g['nb'], body, 0)


def _bn_stats(src_ref, m_ref, g, s, src_ext):
    mean = s / g['m']
    ss = _var_pass(src_ref, m_ref, g, mean, src_ext)
    return mean, jnp.sqrt(ss / g['m'] + _EPS)


def _stem_body(g, x_ref, w_ref, m_ref, o_ref, c_ref):
    s = _conv_pass(x_ref, w_ref, m_ref, c_ref, g, w_ref.shape[0], False)
    mean, r = _bn_stats(c_ref, m_ref, g, s, False)
    _norm_pass(c_ref, m_ref, o_ref, g, mean, r, True, False, False)


def _res_body(g, x_ref, wa_ref, wb_ref, m_ref, o_ref, h_ref, c_ref):
    co = wb_ref.shape[-1]
    _zero_margins(h_ref, g, co)
    _zero_margins(o_ref, g, co)
    s = _conv_pass(x_ref, wa_ref, m_ref, h_ref, g, 3, True)
    mean, r = _bn_stats(h_ref, m_ref, g, s, True)
    _norm_pass(h_ref, m_ref, h_ref, g, mean, r, True, True, True)
    s2 = _conv_pass(h_ref, wb_ref, m_ref, c_ref, g, 3, False)
    mean2, r2 = _bn_stats(c_ref, m_ref, g, s2, False)
    _norm_pass(c_ref, m_ref, o_ref, g, mean2, r2, True, False, True,
               add_ref=x_ref, add_ext=True)


def _res_proj_body(g, x_ref, wa_ref, wb_ref, wd_ref, m_ref, o_ref,
                   h_ref, c_ref, p_ref):
    co = wb_ref.shape[-1]
    _zero_margins(h_ref, g, co)
    _zero_margins(o_ref, g, co)
    s = _conv_pass(x_ref, wa_ref, m_ref, h_ref, g, 3, True)
    mean, r = _bn_stats(h_ref, m_ref, g, s, True)
    _norm_pass(h_ref, m_ref, h_ref, g, mean, r, True, True, True)
    s2 = _conv_pass(h_ref, wb_ref, m_ref, c_ref, g, 3, False)
    mean2, r2 = _bn_stats(c_ref, m_ref, g, s2, False)
    sp = _mm_pass(x_ref, wd_ref, m_ref, p_ref, g, True)
    meanp, rp = _bn_stats(p_ref, m_ref, g, sp, False)
    _norm_pass(c_ref, m_ref, o_ref, g, mean2, r2, True, False, True,
               add_ref=p_ref, add_stats=(meanp, rp))


def _down_body(w_ref, *refs):
    xrefs, o_ref = refs[:8], refs[8]
    acc = None
    for i, (a, b, c) in enumerate(itertools.product(range(2), repeat=3)):
        t = jnp.dot(xrefs[i][...], w_ref[a, b, c],
                    preferred_element_type=jnp.float32)
        acc = t if acc is None else acc + t
    m = jnp.mean(acc, axis=0, keepdims=True)
    cen = acc - m
    v = jnp.mean(cen * cen, axis=0, keepdims=True)
    o_ref[...] = jnp.maximum(cen / jnp.sqrt(v + _EPS), 0.0)


def _up_body(g, x_ref, w_ref, m_ref, o_ref, t_ref):
    blk, e0, mpr = g['blk'], g['e0'], g['mpr']
    co = w_ref.shape[-1]
    taps = list(itertools.product(range(2), repeat=3))

    def body(i, carry):
        s, comp = carry
        base = i * blk
        x = x_ref[pl.ds(e0 + base, blk), :]
        for j, (a, b, c) in enumerate(taps):
            t = jnp.dot(x, w_ref[1 - a, 1 - b, 1 - c],
                        preferred_element_type=jnp.float32)
            t_ref[pl.ds(j * mpr + base, blk), :] = t
            y = jnp.sum(t, axis=0, keepdims=True) - comp
            u = s + y
            comp = (u - s) - y
            s = u
        return s, comp

    z = jnp.zeros((1, co), jnp.float32)
    s = jax.lax.fori_loop(0, g['nb'], body, (z, z))[0]
    mean = s / (8 * g['m'])

    def bodyv(i, carry):
        ss, comp = carry
        base = i * blk
        mt = m_ref[pl.ds(base, blk), :]
        for j in range(8):
            cen = (t_ref[pl.ds(j * mpr + base, blk), :] - mean) * mt
            y = jnp.sum(cen * cen, axis=0, keepdims=True) - comp
            u = ss + y
            comp = (u - ss) - y
            ss = u
        return ss, comp

    ss = jax.lax.fori_loop(0, g['nb'], bodyv, (z, z))[0]
    r = jnp.sqrt(ss / (8 * g['m']) + _EPS)

    def body2(i, _):
        base = i * blk
        mt = m_ref[pl.ds(base, blk), :]
        for j in range(8):
            t = t_ref[pl.ds(j * mpr + base, blk), :]
            o_ref[pl.ds(j * mpr + base, blk), :] = jnp.maximum(
                (t - mean) / r * mt, 0.0)
        return 0

    jax.lax.fori_loop(0, g['nb'], body2, 0)


def _cls_body(x_ref, w_ref, o_ref):
    nb = x_ref.shape[0] // 512

    def body(i, _):
        o_ref[pl.ds(i * 512, 512), :] = jnp.dot(
            x_ref[pl.ds(i * 512, 512), :], w_ref[...][0, 0, 0],
            preferred_element_type=jnp.float32)
        return 0

    jax.lax.fori_loop(0, nb, body, 0)


def _vmem(rows, c):
    return pltpu.VMEM((rows, c), jnp.float32)


def _call(body, out_rows, out_c, scratch, *args):
    return pl.pallas_call(
        body,
        out_shape=jax.ShapeDtypeStruct((out_rows, out_c), jnp.float32),
        scratch_shapes=scratch,
    )(*args)


def _down(x5, w):
    subs = [x5[:, a::2, b::2, c::2, :].reshape(-1, x5.shape[-1])
            for a, b, c in itertools.product(range(2), repeat=3)]
    m2 = subs[0].shape[0]
    return _call(_down_body, m2, w.shape[-1], [], w, *subs)


def _res_chain(xf, p, s, g):
    mask = _mask_const(g)
    ext = 2 * g['e0'] + g['mpr'] + 8
    c1 = p[f'{s}_r1b'].shape[-1]
    if f'{s}_r1d' in p:
        body = lambda *r: _res_proj_body(g, *r)
        xf = _call(body, ext, c1,
                   [_vmem(ext, c1), _vmem(g['mpr'], c1), _vmem(g['mpr'], c1)],
                   xf, p[f'{s}_r1a'], p[f'{s}_r1b'], p[f'{s}_r1d'], mask)
    else:
        body = lambda *r: _res_body(g, *r)
        xf = _call(body, ext, c1, [_vmem(ext, c1), _vmem(g['mpr'], c1)],
                   xf, p[f'{s}_r1a'], p[f'{s}_r1b'], mask)
    body2 = lambda *r: _res_body(g, *r)
    xf = _call(body2, ext, c1, [_vmem(ext, c1), _vmem(g['mpr'], c1)],
               xf, p[f'{s}_r2a'], p[f'{s}_r2b'], mask)
    return xf


def _up(xf, w, g):
    co = w.shape[-1]
    mpr = g['mpr']
    body = lambda *r: _up_body(g, *r)
    out = _call(body, 8 * mpr, co, [_vmem(8 * mpr, co)],
                xf, w, _mask_const(g))
    r = out.reshape(8, mpr, co)[:, :g['mp'], :]
    r = r.reshape(2, 2, 2, g['n'], g['dp'], g['hp'], g['wp'], co)
    r = r.transpose(3, 4, 0, 5, 1, 6, 2, 7)
    r = r.reshape(g['n'], 2 * g['dp'], 2 * g['hp'], 2 * g['wp'], co)
    p = g['p']
    return r[:, 2 * p:2 * p + 2 * g['d'], 2 * p:2 * p + 2 * g['h'],
             2 * g['pw0']:2 * g['pw0'] + 2 * g['w'], :]


@jax.jit
def kernel(x, params):
    p = params
    n, d, h, w, _ = x.shape
    gs = _geom(n, d, h, w, 2)
    g16 = _geom(n, d, h, w, 1)
    g8 = _geom(n, d // 2, h // 2, w // 2, 1)
    g4 = _geom(n, d // 4, h // 4, w // 4, 1)
    g2 = _geom(n, d // 8, h // 8, w // 8, 1)
    g1 = _geom(n, d // 16, h // 16, w // 16, 1)

    c0 = p['stem'].shape[-1]
    sbody = lambda *r: _stem_body(gs, *r)
    x0g = _call(sbody, gs['mpr'], c0, [_vmem(gs['mpr'], c0)],
                _pad_flat_ext(x, gs), p['stem'], _mask_const(gs))
    x0 = x0g[:gs['mp']].reshape(n, gs['dp'], gs['hp'], gs['wp'], c0)[
        :, 2:2 + d, 2:2 + h, gs['pw0']:gs['pw0'] + w, :]

    y = _down(x0, p['s1_down'])
    x1g = _res_chain(_pad_flat_ext(
        y.reshape(n, d // 2, h // 2, w // 2, -1), g8), p, 's1', g8)
    x1 = _unpad5_ext(x1g, g8, x1g.shape[-1])

    y = _down(x1, p['s2_down'])
    x2g = _res_chain(_pad_flat_ext(
        y.reshape(n, d // 4, h // 4, w // 4, -1), g4), p, 's2', g4)
    x2 = _unpad5_ext(x2g, g4, x2g.shape[-1])

    y = _down(x2, p['s3_down'])
    x3g = _res_chain(_pad_flat_ext(
        y.reshape(n, d // 8, h // 8, w // 8, -1), g2), p, 's3', g2)
    x3 = _unpad5_ext(x3g, g2, x3g.shape[-1])

    y = _down(x3, p['s4_down'])
    x4g = _res_chain(_pad_flat_ext(
        y.reshape(n, d // 16, h // 16, w // 16, -1), g1), p, 's4', g1)

    u = _up(x4g, p['u1_de'], g1)
    yg = _res_chain(_pad_flat_ext(jnp.concatenate([u, x3], -1), g2),
                    p, 'u1', g2)
    u = _up(yg, p['u2_de'], g2)
    yg = _res_chain(_pad_flat_ext(jnp.concatenate([u, x2], -1), g4),
                    p, 'u2', g4)
    u = _up(yg, p['u3_de'], g4)
    yg = _res_chain(_pad_flat_ext(jnp.concatenate([u, x1], -1), g8),
                    p, 'u3', g8)
    u = _up(yg, p['u4_de'], g8)
    yg = _res_chain(_pad_flat_ext(jnp.concatenate([u, x0], -1), g16),
                    p, 'u4', g16)

    yf = _unpad5_ext(yg, g16, yg.shape[-1]).reshape(-1, yg.shape[-1])
    return _call(_cls_body, yf.shape[0], p['cls'].shape[-1], [],
                 yf, p['cls'])
